# baseline (speedup 1.0000x reference)
import functools

import jax
import jax.numpy as jnp
from jax import lax
from jax.experimental import pallas as pl
from jax.experimental.pallas import tpu as pltpu
from jax.experimental.pallas import tpu_sc as plsc

_NC, _NS, _L = 2, 16, 16
_NW = _NC * _NS


@functools.cache
def _make_sc_loss(n: int, c: int, d: int):
    num_rows = n * c
    assert num_rows % (_NW * _L) == 0
    rows_per_w = num_rows // _NW
    chunks = rows_per_w // _L
    cpr = c // _L
    rpw = rows_per_w // c
    assert cpr >= 1 and rows_per_w % c == 0
    mesh = plsc.VectorSubcoreMesh(core_axis_name="c", subcore_axis_name="s")

    @functools.partial(
        pl.kernel,
        out_type=jax.ShapeDtypeStruct((_NW, _L), jnp.float32),
        mesh=mesh,
        compiler_params=pltpu.CompilerParams(
            needs_layout_passes=False,
            disable_bounds_checks=True,
            disable_semaphore_checks=True,
            skip_device_barrier=True,
        ),
        scratch_types=[
            pltpu.VMEM((8, c), jnp.int32),
            pltpu.VMEM((8, c), jnp.float32),
            pltpu.VMEM((rows_per_w, 8, 128), jnp.float32),
            pltpu.VMEM((_L,), jnp.float32),
            pltpu.SemaphoreType.DMA,
        ],
    )
    def k(prob_hbm, tgt_hbm, rew_hbm, out_hbm, tgt_v, rew_v, val_v,
          acc_v, sem):
        wid = lax.axis_index("s") * _NC + lax.axis_index("c")
        base = wid * rows_per_w
        blk = pl.multiple_of((wid * rpw // 8) * 8, 8)
        r0 = wid * rpw - blk
        pltpu.sync_copy(tgt_hbm.at[pl.ds(blk, 8)], tgt_v)
        pltpu.sync_copy(rew_hbm.at[pl.ds(blk, 8)], rew_v)
        lane = lax.iota(jnp.int32, _L)

        def chunk_of(ref, j):
            return ref[r0 + j // cpr, pl.ds((j % cpr) * _L, _L)]

        copies = []
        for j in range(chunks):
            t_chunk = chunk_of(tgt_v, j)
            for l in range(_L):
                i = j * _L + l
                cb = pl.multiple_of((t_chunk[l] >> 7) << 7, 128)
                rb = pl.multiple_of(base + 8 * (i // 8), 8)
                copies.append(pltpu.async_copy(
                    prob_hbm.at[pl.ds(rb, 8), pl.ds(cb, 128)],
                    val_v.at[i], sem))
        for cp in copies:
            cp.wait()
        acc = jnp.zeros((_L,), jnp.float32)
        for j in range(chunks):
            ivec = j * _L + lane
            rvec = lane & 7
            cvec = chunk_of(tgt_v, j) & 127
            vals = plsc.load_gather(val_v, [ivec, rvec, cvec])
            acc = acc + vals * chunk_of(rew_v, j)
        acc_v[...] = acc
        pltpu.sync_copy(acc_v, out_hbm.at[wid])

    return k


def kernel(prob, target, reward):
    _, d = prob.shape
    n, c = target.shape
    partials = _make_sc_loss(n, c, d)(
        prob, target.astype(jnp.int32), reward)
    return -jnp.sum(partials)


# speedup vs baseline: 1.0097x; 1.0097x over previous
import functools

import jax
import jax.numpy as jnp
from jax import lax
from jax.experimental import pallas as pl
from jax.experimental.pallas import tpu as pltpu
from jax.experimental.pallas import tpu_sc as plsc

_NC, _NS, _L = 2, 16, 16
_NW = _NC * _NS


@functools.cache
def _make_sc_loss(n: int, c: int, d: int):
    num_rows = n * c
    assert num_rows % (_NW * _L) == 0
    rows_per_w = num_rows // _NW
    chunks = rows_per_w // _L
    cpr = c // _L
    rpw = rows_per_w // c
    assert cpr >= 1 and rows_per_w % c == 0
    mesh = plsc.VectorSubcoreMesh(core_axis_name="c", subcore_axis_name="s")

    @functools.partial(
        pl.kernel,
        out_type=jax.ShapeDtypeStruct((_NW, _L), jnp.float32),
        mesh=mesh,
        compiler_params=pltpu.CompilerParams(
            needs_layout_passes=False,
            disable_bounds_checks=True,
            disable_semaphore_checks=True,
            skip_device_barrier=True,
        ),
        scratch_types=[
            pltpu.VMEM((8, c), jnp.int32),
            pltpu.VMEM((8, c), jnp.float32),
            pltpu.VMEM((rows_per_w * 8, 128), jnp.float32),
            pltpu.VMEM((_L,), jnp.float32),
            pltpu.SemaphoreType.DMA,
        ],
    )
    def k(prob_hbm, tgt_hbm, rew_hbm, out_hbm, tgt_v, rew_v, val_v,
          acc_v, sem):
        wid = lax.axis_index("s") * _NC + lax.axis_index("c")
        base = wid * rows_per_w
        blk = pl.multiple_of((wid * rpw // 8) * 8, 8)
        r0 = wid * rpw - blk
        pltpu.sync_copy(tgt_hbm.at[pl.ds(blk, 8)], tgt_v)
        pltpu.sync_copy(rew_hbm.at[pl.ds(blk, 8)], rew_v)
        lane = lax.iota(jnp.int32, _L)

        def chunk_of(ref, j):
            return ref[r0 + j // cpr, pl.ds((j % cpr) * _L, _L)]

        for j in range(chunks):
            t_chunk = chunk_of(tgt_v, j)
            for l in range(_L):
                i = j * _L + l
                cb = pl.multiple_of((t_chunk[l] >> 7) << 7, 128)
                rb = pl.multiple_of(base + 8 * (i // 8), 8)
                pltpu.async_copy(
                    prob_hbm.at[pl.ds(rb, 8), pl.ds(cb, 128)],
                    val_v.at[pl.ds(i * 8, 8)], sem)
        pltpu.make_async_copy(
            prob_hbm.at[pl.ds(0, rows_per_w * 8), pl.ds(0, 128)],
            val_v, sem).wait()
        acc = jnp.zeros((_L,), jnp.float32)
        for j in range(chunks):
            ivec = j * _L + lane
            rvec = lane & 7
            cvec = chunk_of(tgt_v, j) & 127
            vals = plsc.load_gather(val_v, [ivec * 8 + rvec, cvec])
            acc = acc + vals * chunk_of(rew_v, j)
        acc_v[...] = acc
        pltpu.sync_copy(acc_v, out_hbm.at[wid])

    return k


def kernel(prob, target, reward):
    _, d = prob.shape
    n, c = target.shape
    partials = _make_sc_loss(n, c, d)(
        prob, target.astype(jnp.int32), reward)
    return -jnp.sum(partials)


# speedup vs baseline: 1.0365x; 1.0266x over previous
import functools

import jax
import jax.numpy as jnp
from jax import lax
from jax.experimental import pallas as pl
from jax.experimental.pallas import tpu as pltpu
from jax.experimental.pallas import tpu_sc as plsc

_NC, _NS, _L = 2, 16, 16
_NW = _NC * _NS


@functools.cache
def _make_sc_loss(n: int, c: int, d: int):
    num_rows = n * c
    assert num_rows % (_NW * _L) == 0
    rows_per_w = num_rows // _NW
    chunks = rows_per_w // _L
    cpr = c // _L
    rpw = rows_per_w // c
    assert cpr >= 1 and rows_per_w % c == 0
    mesh = plsc.VectorSubcoreMesh(core_axis_name="c", subcore_axis_name="s")

    @functools.partial(
        pl.kernel,
        out_type=jax.ShapeDtypeStruct((_NW, _L), jnp.float32),
        mesh=mesh,
        compiler_params=pltpu.CompilerParams(
            needs_layout_passes=False,
            disable_bounds_checks=True,
            disable_semaphore_checks=True,
            skip_device_barrier=True,
        ),
        scratch_types=[
            pltpu.VMEM((8, c), jnp.int32),
            pltpu.VMEM((8, c), jnp.float32),
            pltpu.VMEM((rows_per_w * 8, 128), jnp.float32),
            pltpu.VMEM((_L,), jnp.float32),
            pltpu.SemaphoreType.DMA,
            pltpu.SemaphoreType.DMA,
        ],
    )
    def k(prob_hbm, tgt_hbm, rew_hbm, out_hbm, tgt_v, rew_v, val_v,
          acc_v, sem, sem2):
        wid = lax.axis_index("s") * _NC + lax.axis_index("c")
        base = wid * rows_per_w
        blk = pl.multiple_of((wid * rpw // 8) * 8, 8)
        r0 = wid * rpw - blk
        tgt_cp = pltpu.async_copy(tgt_hbm.at[pl.ds(blk, 8)], tgt_v, sem)
        rew_cp = pltpu.async_copy(rew_hbm.at[pl.ds(blk, 8)], rew_v, sem2)
        tgt_cp.wait()
        lane = lax.iota(jnp.int32, _L)

        def chunk_of(ref, j):
            return ref[r0 + j // cpr, pl.ds((j % cpr) * _L, _L)]

        for j in range(chunks):
            t_chunk = chunk_of(tgt_v, j)
            for l in range(_L):
                i = j * _L + l
                cb = pl.multiple_of((t_chunk[l] >> 7) << 7, 128)
                rb = pl.multiple_of(base + 8 * (i // 8), 8)
                pltpu.async_copy(
                    prob_hbm.at[pl.ds(rb, 8), pl.ds(cb, 128)],
                    val_v.at[pl.ds(i * 8, 8)], sem)
        rew_cp.wait()
        pltpu.make_async_copy(
            prob_hbm.at[pl.ds(0, rows_per_w * 8), pl.ds(0, 128)],
            val_v, sem).wait()
        acc = jnp.zeros((_L,), jnp.float32)
        for j in range(chunks):
            ivec = j * _L + lane
            rvec = lane & 7
            cvec = chunk_of(tgt_v, j) & 127
            vals = plsc.load_gather(val_v, [ivec * 8 + rvec, cvec])
            acc = acc + vals * chunk_of(rew_v, j)
        acc_v[...] = acc
        pltpu.sync_copy(acc_v, out_hbm.at[wid])

    return k


def kernel(prob, target, reward):
    _, d = prob.shape
    n, c = target.shape
    partials = _make_sc_loss(n, c, d)(
        prob, target.astype(jnp.int32), reward)
    return -jnp.sum(partials)
